# baseline (speedup 1.0000x reference)
import jax
import jax.numpy as jnp
from jax import lax
from jax.experimental import pallas as pl
from jax.experimental.pallas import tpu as pltpu
from jax.experimental.pallas import tpu_sc as plsc

NU = 50000
NN = 100000
NE = 600000
NEP = 614400
NMR = 600
HP = 50048
ACCR = 51200
D = 64
NSL = 4
B = 4096
MAGIC = 0x5F3759DF

_mesh = plsc.VectorSubcoreMesh(core_axis_name="c", subcore_axis_name="s")
_cparams = pltpu.CompilerParams(
    needs_layout_passes=False, use_tc_tiling_on_sc=False)

_f32 = jnp.float32
_i32 = jnp.int32


def _z16i(v):
  return jnp.zeros((16,), _i32) + v


def _rsqrt16(x):
  y = plsc.bitcast(MAGIC - (plsc.bitcast(x, _i32) >> 1), _f32)
  for _ in range(3):
    y = y * (1.5 - 0.5 * x * y * y)
  return y


def _k0_body(both, d216, dsq16, y,
             hist, idxmb, tmps, degst, d2v, dsv, yv, exb, sbh):
  cid = lax.axis_index("c")
  tid = lax.axis_index("s")
  zeros16 = jnp.zeros((16,), _f32)
  ones16 = jnp.ones((16,), _f32)

  def _zh(i, _):
    hist[pl.ds(i * 16, 16)] = zeros16
  lax.fori_loop(0, NU // 16, _zh, None)

  nmr_a = jnp.where(tid == 15, 30, 38)

  def _dega_mr(i, _):
    pltpu.sync_copy(both.at[cid].at[pl.ds(38 * tid + i, 1)], idxmb)
    def _row(r, _):
      for k in range(8):
        v = idxmb[0, r, pl.ds(k * 16, 16)]
        plsc.addupdate_scatter(hist, [v], ones16, mask=v < NU)
      return None
    lax.fori_loop(0, 8, _row, None)
    return None
  lax.fori_loop(0, nmr_a, _dega_mr, None)

  base = 3200 * tid
  def _zds(i, _):
    degst[pl.ds(i * 16, 16)] = zeros16
  lax.fori_loop(0, 200, _zds, None)
  SLOT = 51200
  for wave in range(2):
    @pl.when((tid >= 8 * wave) & (tid < 8 * wave + 8))
    def _():
      pltpu.sync_copy(hist, sbh.at[pl.ds((tid - 8 * wave) * SLOT, NU)])
    plsc.subcore_barrier()
    for u in range(8):
      pltpu.sync_copy(sbh.at[pl.ds(u * SLOT + base, 3200)], tmps)
      def _acc(i, _):
        degst[pl.ds(i * 16, 16)] = degst[pl.ds(i * 16, 16)] + tmps[pl.ds(i * 16, 16)]
      lax.fori_loop(0, 200, _acc, None)
    plsc.subcore_barrier()

  def _phase_b(nblk):

    def _blk(blk, _):
      r0 = base + blk * 400
      def _nw(i, _):
        x = degst[pl.ds(blk * 400 + i * 16, 16)] + 1e-9
        yy = _rsqrt16(x)
        yv[pl.ds(i * 16, 16)] = yy
        d2v[pl.ds(i * 16, 16)] = yy * yy
        dsv[pl.ds(i * 16, 16)] = x * yy
        return None
      lax.fori_loop(0, 25, _nw, None)
      for sub in range(5):
        def _ex(r, _):
          exb[r, :] = plsc.load_gather(d2v, [_z16i(sub * 80 + r)])
          return None
        lax.fori_loop(0, 80, _ex, None)
        pltpu.sync_copy(exb, d216.at[cid].at[pl.ds(r0 + sub * 80, 80)])
        def _ex2(r, _):
          exb[r, :] = plsc.load_gather(dsv, [_z16i(sub * 80 + r)])
          return None
        lax.fori_loop(0, 80, _ex2, None)
        pltpu.sync_copy(exb, dsq16.at[cid].at[pl.ds(r0 + sub * 80, 80)])
      pltpu.sync_copy(yv, y.at[cid].at[pl.ds(r0, 400)])
      return None
    lax.fori_loop(0, nblk, _blk, None)

  @pl.when(tid < 15)
  def _():
    _phase_b(8)

  @pl.when(tid == 15)
  def _():
    _phase_b(5)


_k0 = pl.kernel(
    _k0_body,
    out_type=(
        jax.ShapeDtypeStruct((2, NU, 16), _f32),
        jax.ShapeDtypeStruct((2, NU, 16), _f32),
        jax.ShapeDtypeStruct((2, NU), _f32),
    ),
    mesh=_mesh,
    compiler_params=_cparams,
    scratch_types=[
        pltpu.VMEM((NU,), _f32),
        pltpu.VMEM((1, 8, 128), _i32),
        pltpu.VMEM((3200,), _f32),
        pltpu.VMEM((3200,), _f32),
        pltpu.VMEM((400,), _f32),
        pltpu.VMEM((400,), _f32),
        pltpu.VMEM((400,), _f32),
        pltpu.VMEM((80, 16), _f32),
        pltpu.VMEM_SHARED((8 * 51200,), _f32),
    ],
)


def _k1b_body(e0, y, p0, yst, e0b, pb0, pb1, pb2, pb3):
  cid = lax.axis_index("c")
  tid = lax.axis_index("s")
  base = 3200 * tid
  pbufs = (pb0, pb1, pb2, pb3)

  def _phase_c(nblk):
    pltpu.sync_copy(y.at[cid].at[pl.ds(base, nblk * 80)],
                    yst.at[pl.ds(0, nblk * 80)])

    def _cblk(blk, _):
      r0 = base + blk * 80
      pltpu.sync_copy(e0.at[pl.ds(NU * cid + r0, 80)], e0b)
      def _row(r, _):
        sy = plsc.load_gather(yst, [_z16i(blk * 80 + r)])
        for s in range(4):
          pbufs[s][r, :] = e0b[r, pl.ds(s * 16, 16)] * sy
        return None
      lax.fori_loop(0, 80, _row, None)
      for s in range(4):
        pltpu.sync_copy(pbufs[s], p0.at[s].at[cid].at[pl.ds(r0, 80)])
      return None
    lax.fori_loop(0, nblk, _cblk, None)

  @pl.when(tid < 15)
  def _():
    _phase_c(40)

  @pl.when(tid == 15)
  def _():
    _phase_c(25)


_k1b = pl.kernel(
    _k1b_body,
    out_type=jax.ShapeDtypeStruct((NSL, 2, HP, 16), _f32),
    mesh=_mesh,
    compiler_params=_cparams,
    scratch_types=[
        pltpu.VMEM((3200,), _f32),
        pltpu.VMEM((80, 64), _f32),
        pltpu.VMEM((80, 16), _f32),
        pltpu.VMEM((80, 16), _f32),
        pltpu.VMEM((80, 16), _f32),
        pltpu.VMEM((80, 16), _f32),
    ],
)


def _k2_body(p, both, d216, pn,
             idxd, idxs, gb0, gb1, gb2, gb3, gb4, gb5, wb, d2b, acc, psl,
             sg0, sg1, sg2, sg3, sg4, sg5, ss0, ss1, ss2, ss3, ss4, ss5):
  cid = lax.axis_index("c")
  tid = lax.axis_index("s")
  zeros16 = jnp.zeros((16,), _f32)
  gbufs = (gb0, gb1, gb2, gb3, gb4, gb5)
  semsg = (sg0, sg1, sg2, sg3, sg4, sg5)
  semss = (ss0, ss1, ss2, ss3, ss4, ss5)

  for pss in range(2):
    sl = 2 * cid + pss
    for h in range(2):
      pltpu.sync_copy(p.at[sl].at[1 - h].at[pl.ds(3128 * tid, 3128)],
                      psl.at[pl.ds(3128 * tid, 3128)])
      def _zz(r, _):
        gb0[r, :] = zeros16
      lax.fori_loop(0, 128, _zz, None)
      def _za(k, _):
        pltpu.sync_copy(gb0, acc.at[pl.ds(3200 * tid + 128 * k, 128)])
      lax.fori_loop(0, 25, _za, None)
      plsc.subcore_barrier()

      dstref = both.at[h]
      srcref = both.at[1 - h]
      nsb = jnp.where(tid < 4, 7, 6)

      def _sb(sbi, _):
        row0 = 6 * (tid + 16 * sbi)
        pltpu.sync_copy(dstref.at[pl.ds(row0, 6)], idxd)
        pltpu.sync_copy(srcref.at[pl.ds(row0, 6)], idxs)
        NCH = 48
        gds = [None] * NCH
        sds = [None] * NCH
        for j in range(NCH):
          b = j % 6
          if j >= 6:
            sds[j - 6].wait()
          gds[j] = pltpu.async_copy(
              psl.at[idxs.at[j // 8, j % 8]], gbufs[b], semsg[b])
          if j >= 1:
            bp = (j - 1) % 6
            gds[j - 1].wait()
            sds[j - 1] = pltpu.async_copy(
                gbufs[bp], acc.at[idxd.at[(j - 1) // 8, (j - 1) % 8]],
                semss[bp], add=True)
        gds[NCH - 1].wait()
        sds[NCH - 1] = pltpu.async_copy(
            gbufs[(NCH - 1) % 6], acc.at[idxd.at[5, 7]],
            semss[(NCH - 1) % 6], add=True)
        for j in range(NCH - 6, NCH):
          sds[j].wait()
        return None
      lax.fori_loop(0, nsb, _sb, None)
      plsc.subcore_barrier()

      def _wout(nblk):
        def _blk(blk, _):
          r0 = 3200 * tid + 80 * blk
          pltpu.sync_copy(acc.at[pl.ds(r0, 80)], wb)
          pltpu.sync_copy(d216.at[h].at[pl.ds(r0, 80)], d2b)
          def _row(r, _):
            wb[r, :] = wb[r, :] * d2b[r, :]
            return None
          lax.fori_loop(0, 80, _row, None)
          pltpu.sync_copy(wb, pn.at[sl].at[h].at[pl.ds(r0, 80)])
          return None
        lax.fori_loop(0, nblk, _blk, None)

      @pl.when(tid < 15)
      def _():
        _wout(40)

      @pl.when(tid == 15)
      def _():
        _wout(25)


_k2 = pl.kernel(
    _k2_body,
    out_type=jax.ShapeDtypeStruct((NSL, 2, HP, 16), _f32),
    mesh=_mesh,
    compiler_params=_cparams,
    scratch_types=[
        pltpu.VMEM((6, 8, 128), _i32),
        pltpu.VMEM((6, 8, 128), _i32),
        pltpu.VMEM((128, 16), _f32),
        pltpu.VMEM((128, 16), _f32),
        pltpu.VMEM((128, 16), _f32),
        pltpu.VMEM((128, 16), _f32),
        pltpu.VMEM((128, 16), _f32),
        pltpu.VMEM((128, 16), _f32),
        pltpu.VMEM((80, 16), _f32),
        pltpu.VMEM((80, 16), _f32),
        pltpu.VMEM_SHARED((ACCR, 16), _f32),
        pltpu.VMEM_SHARED((HP, 16), _f32),
    ] + [pltpu.SemaphoreType.DMA] * 12,
)


def _k3_body(p0, p1, p2, p3, dsq16, xall, out,
             xb, dbuf, g0, g1, g2, g3, obuf, semd, sem0, sem1, sem2, sem3):
  cid = lax.axis_index("c")
  tid = lax.axis_index("s")
  w = tid * 2 + cid
  h = w // 16
  pltpu.sync_copy(xall.at[pl.ds(w // 4, 1)], xb)
  r0 = 2 * w % 8

  ps = (p0, p1, p2, p3)
  gs = (g0, g1, g2, g3)
  sems = (sem0, sem1, sem2, sem3)
  for j in range(2):
    idxr = xb.at[0, r0 + j]
    pltpu.async_copy(dsq16.at[h].at[idxr], dbuf, semd).wait()
    def _scl(r, _):
      dbuf[r, :] = dbuf[r, :] * 0.25
      return None
    lax.fori_loop(0, 128, _scl, None)
    for s in range(4):
      ds = [pltpu.async_copy(ps[t].at[s].at[h].at[idxr], gs[t], sems[t])
            for t in range(4)]
      for dd in ds:
        dd.wait()
      def _row(r, _):
        v = (g0[r, :] + g1[r, :] + g2[r, :] + g3[r, :]) * dbuf[r, :]
        obuf[r, pl.ds(s * 16, 16)] = v
        return None
      lax.fori_loop(0, 128, _row, None)
    pltpu.sync_copy(obuf, out.at[pl.ds(256 * w + 128 * j, 128)])


_k3 = pl.kernel(
    _k3_body,
    out_type=jax.ShapeDtypeStruct((2 * B, D), _f32),
    mesh=_mesh,
    compiler_params=_cparams,
    scratch_types=[
        pltpu.VMEM((1, 8, 128), _i32),
        pltpu.VMEM((128, 16), _f32),
        pltpu.VMEM((128, 16), _f32),
        pltpu.VMEM((128, 16), _f32),
        pltpu.VMEM((128, 16), _f32),
        pltpu.VMEM((128, 16), _f32),
        pltpu.VMEM((128, 64), _f32),
        pltpu.SemaphoreType.DMA,
        pltpu.SemaphoreType.DMA,
        pltpu.SemaphoreType.DMA,
        pltpu.SemaphoreType.DMA,
        pltpu.SemaphoreType.DMA,
    ],
)


def kernel(user_idx, item_idx, x_user, x_item, E0):
  npad = NEP - NE
  ui = user_idx.astype(_i32)
  ii = item_idx.astype(_i32)
  upad = jnp.concatenate([ui, jnp.full((npad,), NU, _i32)]).reshape(NMR, 8, 128)
  ipad = jnp.concatenate([ii, jnp.full((npad,), NU, _i32)]).reshape(NMR, 8, 128)
  both = jnp.stack([upad, ipad])
  d216, dsq16, y = _k0(both)
  p0 = _k1b(E0, y)
  p1 = _k2(p0, both, d216)
  p2 = _k2(p1, both, d216)
  p3 = _k2(p2, both, d216)
  xall = jnp.concatenate(
      [x_user.astype(_i32), x_item.astype(_i32)]).reshape(8, 8, 128)
  outf = _k3(p0, p1, p2, p3, dsq16, xall)
  return outf.reshape(2, B, D)


# speedup vs baseline: 1.1155x; 1.1155x over previous
import jax
import jax.numpy as jnp
from jax import lax
from jax.experimental import pallas as pl
from jax.experimental.pallas import tpu as pltpu
from jax.experimental.pallas import tpu_sc as plsc

NU = 50000
NN = 100000
NE = 600000
NEP = 614400
NMR = 600
HP = 50048
ACCR = 51200
D = 64
NSL = 4
B = 4096
MAGIC = 0x5F3759DF

_mesh = plsc.VectorSubcoreMesh(core_axis_name="c", subcore_axis_name="s")
_cparams = pltpu.CompilerParams(
    needs_layout_passes=False, use_tc_tiling_on_sc=False)

_f32 = jnp.float32
_i32 = jnp.int32


def _z16i(v):
  return jnp.zeros((16,), _i32) + v


def _rsqrt16(x):
  y = plsc.bitcast(MAGIC - (plsc.bitcast(x, _i32) >> 1), _f32)
  for _ in range(3):
    y = y * (1.5 - 0.5 * x * y * y)
  return y


def _k0_body(both, d216, dsq16, y,
             hist, idxmb, tmps, degst, d2v, dsv, yv, exb, sbh):
  cid = lax.axis_index("c")
  tid = lax.axis_index("s")
  zeros16 = jnp.zeros((16,), _f32)
  ones16 = jnp.ones((16,), _f32)

  def _zh(i, _):
    hist[pl.ds(i * 16, 16)] = zeros16
  lax.fori_loop(0, NU // 16, _zh, None)

  nmr_a = jnp.where(tid == 15, 30, 38)

  def _dega_mr(i, _):
    pltpu.sync_copy(both.at[cid].at[pl.ds(38 * tid + i, 1)], idxmb)
    def _row(r, _):
      for k in range(8):
        v = idxmb[0, r, pl.ds(k * 16, 16)]
        plsc.addupdate_scatter(hist, [v], ones16, mask=v < NU)
      return None
    lax.fori_loop(0, 8, _row, None)
    return None
  lax.fori_loop(0, nmr_a, _dega_mr, None)

  base = 3200 * tid
  def _zds(i, _):
    degst[pl.ds(i * 16, 16)] = zeros16
  lax.fori_loop(0, 200, _zds, None)
  SLOT = 51200
  for wave in range(2):
    @pl.when((tid >= 8 * wave) & (tid < 8 * wave + 8))
    def _():
      pltpu.sync_copy(hist, sbh.at[pl.ds((tid - 8 * wave) * SLOT, NU)])
    plsc.subcore_barrier()
    for u in range(8):
      pltpu.sync_copy(sbh.at[pl.ds(u * SLOT + base, 3200)], tmps)
      def _acc(i, _):
        degst[pl.ds(i * 16, 16)] = degst[pl.ds(i * 16, 16)] + tmps[pl.ds(i * 16, 16)]
      lax.fori_loop(0, 200, _acc, None)
    plsc.subcore_barrier()

  def _phase_b(nblk):

    def _blk(blk, _):
      r0 = base + blk * 400
      def _nw(i, _):
        x = degst[pl.ds(blk * 400 + i * 16, 16)] + 1e-9
        yy = _rsqrt16(x)
        yv[pl.ds(i * 16, 16)] = yy
        d2v[pl.ds(i * 16, 16)] = yy * yy
        dsv[pl.ds(i * 16, 16)] = x * yy
        return None
      lax.fori_loop(0, 25, _nw, None)
      for sub in range(5):
        def _ex(r, _):
          exb[r, :] = plsc.load_gather(d2v, [_z16i(sub * 80 + r)])
          return None
        lax.fori_loop(0, 80, _ex, None)
        pltpu.sync_copy(exb, d216.at[cid].at[pl.ds(r0 + sub * 80, 80)])
        def _ex2(r, _):
          exb[r, :] = plsc.load_gather(dsv, [_z16i(sub * 80 + r)])
          return None
        lax.fori_loop(0, 80, _ex2, None)
        pltpu.sync_copy(exb, dsq16.at[cid].at[pl.ds(r0 + sub * 80, 80)])
      pltpu.sync_copy(yv, y.at[cid].at[pl.ds(r0, 400)])
      return None
    lax.fori_loop(0, nblk, _blk, None)

  @pl.when(tid < 15)
  def _():
    _phase_b(8)

  @pl.when(tid == 15)
  def _():
    _phase_b(5)


_k0 = pl.kernel(
    _k0_body,
    out_type=(
        jax.ShapeDtypeStruct((2, NU, 16), _f32),
        jax.ShapeDtypeStruct((2, NU, 16), _f32),
        jax.ShapeDtypeStruct((2, NU), _f32),
    ),
    mesh=_mesh,
    compiler_params=_cparams,
    scratch_types=[
        pltpu.VMEM((NU,), _f32),
        pltpu.VMEM((1, 8, 128), _i32),
        pltpu.VMEM((3200,), _f32),
        pltpu.VMEM((3200,), _f32),
        pltpu.VMEM((400,), _f32),
        pltpu.VMEM((400,), _f32),
        pltpu.VMEM((400,), _f32),
        pltpu.VMEM((80, 16), _f32),
        pltpu.VMEM_SHARED((8 * 51200,), _f32),
    ],
)


def _k1b_body(e0, y, p0, yst, e0b, pb0, pb1, pb2, pb3):
  cid = lax.axis_index("c")
  tid = lax.axis_index("s")
  base = 3200 * tid
  pbufs = (pb0, pb1, pb2, pb3)

  def _phase_c(nblk):
    pltpu.sync_copy(y.at[cid].at[pl.ds(base, nblk * 80)],
                    yst.at[pl.ds(0, nblk * 80)])

    def _cblk(blk, _):
      r0 = base + blk * 80
      pltpu.sync_copy(e0.at[pl.ds(NU * cid + r0, 80)], e0b)
      def _row(r, _):
        sy = plsc.load_gather(yst, [_z16i(blk * 80 + r)])
        for s in range(4):
          pbufs[s][r, :] = e0b[r, pl.ds(s * 16, 16)] * sy
        return None
      lax.fori_loop(0, 80, _row, None)
      for s in range(4):
        pltpu.sync_copy(pbufs[s], p0.at[s].at[cid].at[pl.ds(r0, 80)])
      return None
    lax.fori_loop(0, nblk, _cblk, None)

  @pl.when(tid < 15)
  def _():
    _phase_c(40)

  @pl.when(tid == 15)
  def _():
    _phase_c(25)


_k1b = pl.kernel(
    _k1b_body,
    out_type=jax.ShapeDtypeStruct((NSL, 2, HP, 16), _f32),
    mesh=_mesh,
    compiler_params=_cparams,
    scratch_types=[
        pltpu.VMEM((3200,), _f32),
        pltpu.VMEM((80, 64), _f32),
        pltpu.VMEM((80, 16), _f32),
        pltpu.VMEM((80, 16), _f32),
        pltpu.VMEM((80, 16), _f32),
        pltpu.VMEM((80, 16), _f32),
    ],
)


def _k2_body(p, both, d216, pn,
             idxd, idxs, gb0, gb1, gb2, gb3, wb0, wb1, d2b0, d2b1, acc, psl,
             sg0, sg1, sg2, sg3, ss0, ss1, ss2, ss3, wi0, wi1, wo0, wo1):
  cid = lax.axis_index("c")
  tid = lax.axis_index("s")
  zeros16 = jnp.zeros((16,), _f32)
  gbufs = (gb0, gb1, gb2, gb3)
  semsg = (sg0, sg1, sg2, sg3)
  semss = (ss0, ss1, ss2, ss3)
  wbs = (wb0, wb1)
  d2bs = (d2b0, d2b1)
  wis = (wi0, wi1)
  wos = (wo0, wo1)

  for pss in range(2):
    sl = 2 * cid + pss
    for h in range(2):
      pltpu.sync_copy(p.at[sl].at[1 - h].at[pl.ds(3128 * tid, 3128)],
                      psl.at[pl.ds(3128 * tid, 3128)])
      def _zz(r, _):
        gb0[r, :] = zeros16
      lax.fori_loop(0, 128, _zz, None)
      zds = [None] * 25
      for k in range(25):
        if k >= 4:
          zds[k - 4].wait()
        zds[k] = pltpu.async_copy(
            gb0, acc.at[pl.ds(3200 * tid + 128 * k, 128)], sg0)
      for k in range(21, 25):
        zds[k].wait()
      plsc.subcore_barrier()

      dstref = both.at[h]
      srcref = both.at[1 - h]
      nsb = jnp.where(tid < 8, 13, 12)

      def _sb(sbi, _):
        row0 = 3 * (tid + 16 * sbi)
        pltpu.sync_copy(dstref.at[pl.ds(row0, 3)], idxd)
        pltpu.sync_copy(srcref.at[pl.ds(row0, 3)], idxs)
        NCH = 24
        gds = [None] * NCH
        sds = [None] * NCH
        for j in range(NCH):
          b = j % 4
          if j >= 4:
            sds[j - 4].wait()
          gds[j] = pltpu.async_copy(
              psl.at[idxs.at[j // 8, j % 8]], gbufs[b], semsg[b])
          if j >= 1:
            bp = (j - 1) % 4
            gds[j - 1].wait()
            sds[j - 1] = pltpu.async_copy(
                gbufs[bp], acc.at[idxd.at[(j - 1) // 8, (j - 1) % 8]],
                semss[bp], add=True)
        gds[NCH - 1].wait()
        sds[NCH - 1] = pltpu.async_copy(
            gbufs[(NCH - 1) % 4], acc.at[idxd.at[2, 7]],
            semss[(NCH - 1) % 4], add=True)
        for j in range(NCH - 4, NCH):
          sds[j].wait()
        return None
      lax.fori_loop(0, nsb, _sb, None)
      plsc.subcore_barrier()

      def _wout(nblk):
        def _blk(blk, _):
          r0 = 3200 * tid + 200 * blk
          pltpu.sync_copy(acc.at[pl.ds(r0, 200)], wb0)
          pltpu.sync_copy(d216.at[h].at[pl.ds(r0, 200)], d2b0)
          def _row(r, _):
            wb0[r, :] = wb0[r, :] * d2b0[r, :]
            return None
          lax.fori_loop(0, 200, _row, None)
          pltpu.sync_copy(wb0, pn.at[sl].at[h].at[pl.ds(r0, 200)])
          return None
        lax.fori_loop(0, nblk, _blk, None)

      @pl.when(tid < 15)
      def _():
        _wout(16)

      @pl.when(tid == 15)
      def _():
        _wout(10)


_k2 = pl.kernel(
    _k2_body,
    out_type=jax.ShapeDtypeStruct((NSL, 2, HP, 16), _f32),
    mesh=_mesh,
    compiler_params=_cparams,
    scratch_types=[
        pltpu.VMEM((3, 8, 128), _i32),
        pltpu.VMEM((3, 8, 128), _i32),
        pltpu.VMEM((128, 16), _f32),
        pltpu.VMEM((128, 16), _f32),
        pltpu.VMEM((128, 16), _f32),
        pltpu.VMEM((128, 16), _f32),
        pltpu.VMEM((200, 16), _f32),
        pltpu.VMEM((200, 16), _f32),
        pltpu.VMEM((200, 16), _f32),
        pltpu.VMEM((200, 16), _f32),
        pltpu.VMEM_SHARED((ACCR, 16), _f32),
        pltpu.VMEM_SHARED((HP, 16), _f32),
    ] + [pltpu.SemaphoreType.DMA] * 12,
)


def _k3_body(p0, p1, p2, p3, dsq16, xall, out,
             xb, dbuf, g0, g1, g2, g3, obuf, semd, sem0, sem1, sem2, sem3):
  cid = lax.axis_index("c")
  tid = lax.axis_index("s")
  w = tid * 2 + cid
  h = w // 16
  pltpu.sync_copy(xall.at[pl.ds(w // 4, 1)], xb)
  r0 = 2 * w % 8

  ps = (p0, p1, p2, p3)
  gs = (g0, g1, g2, g3)
  sems = (sem0, sem1, sem2, sem3)
  for j in range(2):
    idxr = xb.at[0, r0 + j]
    pltpu.async_copy(dsq16.at[h].at[idxr], dbuf, semd).wait()
    def _scl(r, _):
      dbuf[r, :] = dbuf[r, :] * 0.25
      return None
    lax.fori_loop(0, 128, _scl, None)
    for s in range(4):
      ds = [pltpu.async_copy(ps[t].at[s].at[h].at[idxr], gs[t], sems[t])
            for t in range(4)]
      for dd in ds:
        dd.wait()
      def _row(r, _):
        v = (g0[r, :] + g1[r, :] + g2[r, :] + g3[r, :]) * dbuf[r, :]
        obuf[r, pl.ds(s * 16, 16)] = v
        return None
      lax.fori_loop(0, 128, _row, None)
    pltpu.sync_copy(obuf, out.at[pl.ds(256 * w + 128 * j, 128)])


_k3 = pl.kernel(
    _k3_body,
    out_type=jax.ShapeDtypeStruct((2 * B, D), _f32),
    mesh=_mesh,
    compiler_params=_cparams,
    scratch_types=[
        pltpu.VMEM((1, 8, 128), _i32),
        pltpu.VMEM((128, 16), _f32),
        pltpu.VMEM((128, 16), _f32),
        pltpu.VMEM((128, 16), _f32),
        pltpu.VMEM((128, 16), _f32),
        pltpu.VMEM((128, 16), _f32),
        pltpu.VMEM((128, 64), _f32),
        pltpu.SemaphoreType.DMA,
        pltpu.SemaphoreType.DMA,
        pltpu.SemaphoreType.DMA,
        pltpu.SemaphoreType.DMA,
        pltpu.SemaphoreType.DMA,
    ],
)


def kernel(user_idx, item_idx, x_user, x_item, E0):
  npad = NEP - NE
  ui = user_idx.astype(_i32)
  ii = item_idx.astype(_i32)
  upad = jnp.concatenate([ui, jnp.full((npad,), NU, _i32)]).reshape(NMR, 8, 128)
  ipad = jnp.concatenate([ii, jnp.full((npad,), NU, _i32)]).reshape(NMR, 8, 128)
  both = jnp.stack([upad, ipad])
  d216, dsq16, y = _k0(both)
  p0 = _k1b(E0, y)
  p1 = _k2(p0, both, d216)
  p2 = _k2(p1, both, d216)
  p3 = _k2(p2, both, d216)
  xall = jnp.concatenate(
      [x_user.astype(_i32), x_item.astype(_i32)]).reshape(8, 8, 128)
  outf = _k3(p0, p1, p2, p3, dsq16, xall)
  return outf.reshape(2, B, D)


# speedup vs baseline: 1.1398x; 1.0217x over previous
import jax
import jax.numpy as jnp
from jax import lax
from jax.experimental import pallas as pl
from jax.experimental.pallas import tpu as pltpu
from jax.experimental.pallas import tpu_sc as plsc

NU = 50000
NN = 100000
NE = 600000
NEP = 614400
NMR = 600
HP = 50048
ACCR = 51200
D = 64
NSL = 4
B = 4096
MAGIC = 0x5F3759DF

_mesh = plsc.VectorSubcoreMesh(core_axis_name="c", subcore_axis_name="s")
_cparams = pltpu.CompilerParams(
    needs_layout_passes=False, use_tc_tiling_on_sc=False)

_f32 = jnp.float32
_i32 = jnp.int32


def _z16i(v):
  return jnp.zeros((16,), _i32) + v


def _rsqrt16(x):
  y = plsc.bitcast(MAGIC - (plsc.bitcast(x, _i32) >> 1), _f32)
  for _ in range(3):
    y = y * (1.5 - 0.5 * x * y * y)
  return y


def _k0_body(both, d216, dsq16, y,
             hist, idxmb, tmps, degst, d2v, dsv, yv, exb, sbh):
  cid = lax.axis_index("c")
  tid = lax.axis_index("s")
  zeros16 = jnp.zeros((16,), _f32)
  ones16 = jnp.ones((16,), _f32)

  def _zh(i, _):
    hist[pl.ds(i * 16, 16)] = zeros16
  lax.fori_loop(0, NU // 16, _zh, None)

  nmr_a = jnp.where(tid == 15, 30, 38)

  def _dega_mr(i, _):
    pltpu.sync_copy(both.at[cid].at[pl.ds(38 * tid + i, 1)], idxmb)
    def _row(r, _):
      for k in range(8):
        v = idxmb[0, r, pl.ds(k * 16, 16)]
        plsc.addupdate_scatter(hist, [v], ones16, mask=v < NU)
      return None
    lax.fori_loop(0, 8, _row, None)
    return None
  lax.fori_loop(0, nmr_a, _dega_mr, None)

  base = 3200 * tid
  def _zds(i, _):
    degst[pl.ds(i * 16, 16)] = zeros16
  lax.fori_loop(0, 200, _zds, None)
  SLOT = 51200
  for wave in range(2):
    @pl.when((tid >= 8 * wave) & (tid < 8 * wave + 8))
    def _():
      pltpu.sync_copy(hist, sbh.at[pl.ds((tid - 8 * wave) * SLOT, NU)])
    plsc.subcore_barrier()
    for u in range(8):
      pltpu.sync_copy(sbh.at[pl.ds(u * SLOT + base, 3200)], tmps)
      def _acc(i, _):
        degst[pl.ds(i * 16, 16)] = degst[pl.ds(i * 16, 16)] + tmps[pl.ds(i * 16, 16)]
      lax.fori_loop(0, 200, _acc, None)
    plsc.subcore_barrier()

  def _phase_b(nblk):

    def _blk(blk, _):
      r0 = base + blk * 400
      def _nw(i, _):
        x = degst[pl.ds(blk * 400 + i * 16, 16)] + 1e-9
        yy = _rsqrt16(x)
        yv[pl.ds(i * 16, 16)] = yy
        d2v[pl.ds(i * 16, 16)] = yy * yy
        dsv[pl.ds(i * 16, 16)] = x * yy
        return None
      lax.fori_loop(0, 25, _nw, None)
      def _ex(r, _):
        exb[r, :] = plsc.load_gather(d2v, [_z16i(r)])
        return None
      lax.fori_loop(0, 400, _ex, None)
      pltpu.sync_copy(exb, d216.at[cid].at[pl.ds(r0, 400)])
      def _ex2(r, _):
        exb[r, :] = plsc.load_gather(dsv, [_z16i(r)])
        return None
      lax.fori_loop(0, 400, _ex2, None)
      pltpu.sync_copy(exb, dsq16.at[cid].at[pl.ds(r0, 400)])
      pltpu.sync_copy(yv, y.at[cid].at[pl.ds(r0, 400)])
      return None
    lax.fori_loop(0, nblk, _blk, None)

  @pl.when(tid < 15)
  def _():
    _phase_b(8)

  @pl.when(tid == 15)
  def _():
    _phase_b(5)


_k0 = pl.kernel(
    _k0_body,
    out_type=(
        jax.ShapeDtypeStruct((2, NU, 16), _f32),
        jax.ShapeDtypeStruct((2, NU, 16), _f32),
        jax.ShapeDtypeStruct((2, NU), _f32),
    ),
    mesh=_mesh,
    compiler_params=_cparams,
    scratch_types=[
        pltpu.VMEM((NU,), _f32),
        pltpu.VMEM((1, 8, 128), _i32),
        pltpu.VMEM((3200,), _f32),
        pltpu.VMEM((3200,), _f32),
        pltpu.VMEM((400,), _f32),
        pltpu.VMEM((400,), _f32),
        pltpu.VMEM((400,), _f32),
        pltpu.VMEM((400, 16), _f32),
        pltpu.VMEM_SHARED((8 * 51200,), _f32),
    ],
)


def _k1b_body(e0, y, p0, yst, e0b0, e0b1, pb, sie0, sie1, soo0, soo1):
  cid = lax.axis_index("c")
  tid = lax.axis_index("s")
  base = 3200 * tid
  e0bs = (e0b0, e0b1)
  sies = (sie0, sie1)
  soos = (soo0, soo1)

  def _phase_c(nblk):
    pltpu.sync_copy(y.at[cid].at[pl.ds(base, nblk * 200)],
                    yst.at[pl.ds(0, nblk * 200)])
    ins = [None] * nblk
    outs = [None] * nblk

    def _start_in(blk):
      b = blk % 2
      return pltpu.async_copy(
          e0.at[pl.ds(NU * cid + base + 200 * blk, 200)], e0bs[b], sies[b])

    ins[0] = _start_in(0)
    for blk in range(nblk):
      b = blk % 2
      ins[blk].wait()
      if blk + 1 < nblk:
        if blk >= 1:
          for od in outs[blk - 1]:
            od.wait()
        ins[blk + 1] = _start_in(blk + 1)

      def _row(r, _):
        sy = plsc.load_gather(yst, [_z16i(blk * 200 + r)])
        for s in range(4):
          pb[b * 4 + s, r, :] = e0bs[b][r, pl.ds(s * 16, 16)] * sy
        return None
      lax.fori_loop(0, 200, _row, None)
      outs[blk] = [
          pltpu.async_copy(pb.at[b * 4 + s],
                           p0.at[s].at[cid].at[pl.ds(base + 200 * blk, 200)],
                           soos[b])
          for s in range(4)]
    for od in outs[nblk - 1]:
      od.wait()
    if nblk >= 2:
      for od in outs[nblk - 2]:
        od.wait()

  @pl.when(tid < 15)
  def _():
    _phase_c(16)

  @pl.when(tid == 15)
  def _():
    _phase_c(10)


_k1b = pl.kernel(
    _k1b_body,
    out_type=jax.ShapeDtypeStruct((NSL, 2, HP, 16), _f32),
    mesh=_mesh,
    compiler_params=_cparams,
    scratch_types=[
        pltpu.VMEM((3200,), _f32),
        pltpu.VMEM((200, 64), _f32),
        pltpu.VMEM((200, 64), _f32),
        pltpu.VMEM((8, 200, 16), _f32),
        pltpu.SemaphoreType.DMA,
        pltpu.SemaphoreType.DMA,
        pltpu.SemaphoreType.DMA,
        pltpu.SemaphoreType.DMA,
    ],
)


def _k2_body(p, both, d216, pn,
             idxd, idxs, gb0, gb1, gb2, gb3, wb0, wb1, d2b0, d2b1, acc, psl,
             sg0, sg1, sg2, sg3, ss0, ss1, ss2, ss3, wi0, wi1, wo0, wo1):
  cid = lax.axis_index("c")
  tid = lax.axis_index("s")
  zeros16 = jnp.zeros((16,), _f32)
  gbufs = (gb0, gb1, gb2, gb3)
  semsg = (sg0, sg1, sg2, sg3)
  semss = (ss0, ss1, ss2, ss3)
  wbs = (wb0, wb1)
  d2bs = (d2b0, d2b1)
  wis = (wi0, wi1)
  wos = (wo0, wo1)

  for pss in range(2):
    sl = 2 * cid + pss
    for h in range(2):
      pltpu.sync_copy(p.at[sl].at[1 - h].at[pl.ds(3128 * tid, 3128)],
                      psl.at[pl.ds(3128 * tid, 3128)])
      def _zz(r, _):
        gb0[r, :] = zeros16
      lax.fori_loop(0, 128, _zz, None)
      zds = [None] * 25
      for k in range(25):
        if k >= 4:
          zds[k - 4].wait()
        zds[k] = pltpu.async_copy(
            gb0, acc.at[pl.ds(3200 * tid + 128 * k, 128)], sg0)
      for k in range(21, 25):
        zds[k].wait()
      plsc.subcore_barrier()

      dstref = both.at[h]
      srcref = both.at[1 - h]
      nsb = jnp.where(tid < 8, 13, 12)

      def _sb(sbi, _):
        row0 = 3 * (tid + 16 * sbi)
        pltpu.sync_copy(dstref.at[pl.ds(row0, 3)], idxd)
        pltpu.sync_copy(srcref.at[pl.ds(row0, 3)], idxs)
        NCH = 24
        gds = [None] * NCH
        sds = [None] * NCH
        for j in range(NCH):
          b = j % 4
          if j >= 4:
            sds[j - 4].wait()
          gds[j] = pltpu.async_copy(
              psl.at[idxs.at[j // 8, j % 8]], gbufs[b], semsg[b])
          if j >= 1:
            bp = (j - 1) % 4
            gds[j - 1].wait()
            sds[j - 1] = pltpu.async_copy(
                gbufs[bp], acc.at[idxd.at[(j - 1) // 8, (j - 1) % 8]],
                semss[bp], add=True)
        gds[NCH - 1].wait()
        sds[NCH - 1] = pltpu.async_copy(
            gbufs[(NCH - 1) % 4], acc.at[idxd.at[2, 7]],
            semss[(NCH - 1) % 4], add=True)
        for j in range(NCH - 4, NCH):
          sds[j].wait()
        return None
      lax.fori_loop(0, nsb, _sb, None)
      plsc.subcore_barrier()

      def _wout(nblk):
        def _blk(blk, _):
          r0 = 3200 * tid + 200 * blk
          pltpu.sync_copy(acc.at[pl.ds(r0, 200)], wb0)
          pltpu.sync_copy(d216.at[h].at[pl.ds(r0, 200)], d2b0)
          def _row(r, _):
            wb0[r, :] = wb0[r, :] * d2b0[r, :]
            return None
          lax.fori_loop(0, 200, _row, None)
          pltpu.sync_copy(wb0, pn.at[sl].at[h].at[pl.ds(r0, 200)])
          return None
        lax.fori_loop(0, nblk, _blk, None)

      @pl.when(tid < 15)
      def _():
        _wout(16)

      @pl.when(tid == 15)
      def _():
        _wout(10)


_k2 = pl.kernel(
    _k2_body,
    out_type=jax.ShapeDtypeStruct((NSL, 2, HP, 16), _f32),
    mesh=_mesh,
    compiler_params=_cparams,
    scratch_types=[
        pltpu.VMEM((3, 8, 128), _i32),
        pltpu.VMEM((3, 8, 128), _i32),
        pltpu.VMEM((128, 16), _f32),
        pltpu.VMEM((128, 16), _f32),
        pltpu.VMEM((128, 16), _f32),
        pltpu.VMEM((128, 16), _f32),
        pltpu.VMEM((200, 16), _f32),
        pltpu.VMEM((200, 16), _f32),
        pltpu.VMEM((200, 16), _f32),
        pltpu.VMEM((200, 16), _f32),
        pltpu.VMEM_SHARED((ACCR, 16), _f32),
        pltpu.VMEM_SHARED((HP, 16), _f32),
    ] + [pltpu.SemaphoreType.DMA] * 12,
)


def _k3_body(p0, p1, p2, p3, dsq16, xall, out,
             xb, dbuf, g0, g1, g2, g3, obuf, semd, sem0, sem1, sem2, sem3):
  cid = lax.axis_index("c")
  tid = lax.axis_index("s")
  w = tid * 2 + cid
  h = w // 16
  pltpu.sync_copy(xall.at[pl.ds(w // 4, 1)], xb)
  r0 = 2 * w % 8

  ps = (p0, p1, p2, p3)
  gs = (g0, g1, g2, g3)
  sems = (sem0, sem1, sem2, sem3)
  for j in range(2):
    idxr = xb.at[0, r0 + j]
    pltpu.async_copy(dsq16.at[h].at[idxr], dbuf, semd).wait()
    def _scl(r, _):
      dbuf[r, :] = dbuf[r, :] * 0.25
      return None
    lax.fori_loop(0, 128, _scl, None)
    for s in range(4):
      ds = [pltpu.async_copy(ps[t].at[s].at[h].at[idxr], gs[t], sems[t])
            for t in range(4)]
      for dd in ds:
        dd.wait()
      def _row(r, _):
        v = (g0[r, :] + g1[r, :] + g2[r, :] + g3[r, :]) * dbuf[r, :]
        obuf[r, pl.ds(s * 16, 16)] = v
        return None
      lax.fori_loop(0, 128, _row, None)
    pltpu.sync_copy(obuf, out.at[pl.ds(256 * w + 128 * j, 128)])


_k3 = pl.kernel(
    _k3_body,
    out_type=jax.ShapeDtypeStruct((2 * B, D), _f32),
    mesh=_mesh,
    compiler_params=_cparams,
    scratch_types=[
        pltpu.VMEM((1, 8, 128), _i32),
        pltpu.VMEM((128, 16), _f32),
        pltpu.VMEM((128, 16), _f32),
        pltpu.VMEM((128, 16), _f32),
        pltpu.VMEM((128, 16), _f32),
        pltpu.VMEM((128, 16), _f32),
        pltpu.VMEM((128, 64), _f32),
        pltpu.SemaphoreType.DMA,
        pltpu.SemaphoreType.DMA,
        pltpu.SemaphoreType.DMA,
        pltpu.SemaphoreType.DMA,
        pltpu.SemaphoreType.DMA,
    ],
)


def kernel(user_idx, item_idx, x_user, x_item, E0):
  npad = NEP - NE
  ui = user_idx.astype(_i32)
  ii = item_idx.astype(_i32)
  upad = jnp.concatenate([ui, jnp.full((npad,), NU, _i32)]).reshape(NMR, 8, 128)
  ipad = jnp.concatenate([ii, jnp.full((npad,), NU, _i32)]).reshape(NMR, 8, 128)
  both = jnp.stack([upad, ipad])
  d216, dsq16, y = _k0(both)
  p0 = _k1b(E0, y)
  p1 = _k2(p0, both, d216)
  p2 = _k2(p1, both, d216)
  p3 = _k2(p2, both, d216)
  xall = jnp.concatenate(
      [x_user.astype(_i32), x_item.astype(_i32)]).reshape(8, 8, 128)
  outf = _k3(p0, p1, p2, p3, dsq16, xall)
  return outf.reshape(2, B, D)
